# Initial kernel scaffold; baseline (speedup 1.0000x reference)
import jax
import jax.numpy as jnp
from jax.experimental import pallas as pl


def kernel(observations, actions, rewards, dones, truncations, next_observations, obs_data, act_data, next_obs_data, rewards_data, dones_data, truncations_data, ptr, sample_idx):
    raise NotImplementedError("write your pallas kernel here")


# speedup vs baseline: 2.6356x; 2.6356x over previous
import functools

import jax
import jax.numpy as jnp
from jax import lax
from jax.experimental import pallas as pl
from jax.experimental.pallas import tpu as pltpu
from jax.experimental.pallas import tpu_sc as plsc

N_ENV = 64
BUF = 4096
N_OBS = 128
N_ACT = 32
BATCH = 256
OUT_D = N_OBS + N_ACT + N_OBS + 3
DROW_PAD = 304
L = 16
NB = BATCH // L


def _build_kernel(num_cores, num_subcores):
    n_workers = num_cores * num_subcores
    envs_per_worker = N_ENV // n_workers
    mesh = plsc.VectorSubcoreMesh(core_axis_name="c", subcore_axis_name="s")
    f32 = jnp.float32

    @functools.partial(
        pl.kernel,
        out_type=[
            jax.ShapeDtypeStruct((N_ENV * BATCH, N_OBS), f32),
            jax.ShapeDtypeStruct((N_ACT, N_ENV * BATCH), f32),
            jax.ShapeDtypeStruct((N_ENV * BATCH, N_OBS), f32),
            jax.ShapeDtypeStruct((N_ENV * BATCH,), f32),
            jax.ShapeDtypeStruct((N_ENV * BATCH,), f32),
            jax.ShapeDtypeStruct((N_ENV * BATCH,), f32),
        ],
        mesh=mesh,
        compiler_params=pltpu.CompilerParams(needs_layout_passes=False),
        scratch_types=[
            pltpu.VMEM((BATCH,), jnp.int32),
            pltpu.VMEM((128,), jnp.int32),
            pltpu.VMEM((128,), jnp.int32),
            pltpu.VMEM((128,), jnp.int32),
            pltpu.VMEM((128,), jnp.int32),
            pltpu.VMEM((BATCH, N_OBS), f32),
            pltpu.VMEM((BATCH, 128), f32),
            pltpu.VMEM((N_ACT, BATCH), f32),
            pltpu.VMEM((BATCH, N_OBS), f32),
            pltpu.VMEM((BATCH,), f32),
            pltpu.VMEM((BATCH,), f32),
            pltpu.VMEM((BATCH,), f32),
            pltpu.VMEM((BUF,), f32),
            pltpu.VMEM((BUF,), jnp.int32),
            pltpu.VMEM((BUF,), jnp.int32),
            pltpu.VMEM((DROW_PAD,), f32),
            pltpu.VMEM((L,), jnp.int32),
            pltpu.SemaphoreType.DMA,
        ],
    )
    def k(obs_hbm, act_hbm, nobs_hbm, rew_hbm, dn_hbm, tr_hbm, data_hbm,
          tvec_hbm, sidx_hbm,
          o_obs, o_act, o_nobs, o_rew, o_dn, o_tr,
          idx_v, gidx_a, gidx_b, pgidx_a, pgidx_b, obs_stage, act_wide,
          act_stage_t, nobs_stage,
          rew_o, dn_o, tr_o, rew_v, dn_v, tr_v, drow, tv, sem):
        wid = lax.axis_index("s") * num_cores + lax.axis_index("c")
        pltpu.sync_copy(tvec_hbm, tv)
        lane = lax.iota(jnp.int32, L)
        tvec = tv[...]

        for j in range(envs_per_worker):
            e = wid * envs_per_worker + j
            pltpu.sync_copy(sidx_hbm.at[pl.ds(e * BATCH, BATCH)], idx_v)
            pltpu.sync_copy(data_hbm.at[pl.ds(e * DROW_PAD, DROW_PAD)], drow)
            ebase = e * BUF
            for kk in range(NB // 2):
                s = pl.ds(kk * L, L)
                gidx_a[s] = idx_v[s] + ebase
                pgidx_a[s] = lax.shift_right_logical(idx_v[s] + ebase, 2)
            for kk in range(NB // 2):
                s = pl.ds(kk * L, L)
                v = idx_v[pl.ds(128 + kk * L, L)] + ebase
                gidx_b[s] = v
                pgidx_b[s] = lax.shift_right_logical(v, 2)
            copies = []
            for h, (gi, pgi) in enumerate(((gidx_a, pgidx_a),
                                           (gidx_b, pgidx_b))):
                rows = pl.ds(h * 128, 128)
                copies.append(pltpu.async_copy(
                    obs_hbm.at[gi], obs_stage.at[rows], sem))
                copies.append(pltpu.async_copy(
                    act_hbm.at[pgi], act_wide.at[rows], sem))
                copies.append(pltpu.async_copy(
                    nobs_hbm.at[gi], nobs_stage.at[rows], sem))
            pltpu.sync_copy(rew_hbm.at[pl.ds(ebase, BUF)], rew_v)
            pltpu.sync_copy(dn_hbm.at[pl.ds(ebase, BUF)], dn_v)
            pltpu.sync_copy(tr_hbm.at[pl.ds(ebase, BUF)], tr_v)

            def scal_body(kk, _):
                s = pl.ds(kk * L, L)
                ii = idx_v[s]
                rew_o[s] = plsc.load_gather(rew_v, [ii])
                dn_o[s] = plsc.load_gather(dn_v, [ii]).astype(f32)
                tr_o[s] = plsc.load_gather(tr_v, [ii]).astype(f32)
                return 0

            lax.fori_loop(0, NB, scal_body, 0)
            for c in copies:
                c.wait()

            def act_body(kk, _):
                rows16 = kk * L + lane
                ii = idx_v[pl.ds(kk * L, L)]
                off = (ii & 3) * N_ACT

                def act_col(jj, _):
                    vals = plsc.load_gather(act_wide, [rows16, off + jj])
                    plsc.store_scatter(
                        act_stage_t,
                        [jnp.full((L,), jj, jnp.int32), rows16], vals)
                    return 0

                lax.fori_loop(0, N_ACT, act_col, 0)
                return 0

            lax.fori_loop(0, NB, act_body, 0)

            def patch_chunk(kk, _):
                ii = idx_v[pl.ds(kk * L, L)]
                m = (ii == tvec).astype(jnp.int32)
                nm = jnp.sum(m)

                @pl.when(nm > 0)
                def _():
                    def per_lane(l, _):
                        ml = jnp.sum(jnp.where(lane == l, m, 0))

                        @pl.when(ml > 0)
                        def _():
                            b = jnp.full((L,), kk * L + l, jnp.int32)

                            def cp_obs(c, _):
                                cols = c * L + lane
                                plsc.store_scatter(
                                    obs_stage, [b, cols],
                                    plsc.load_gather(drow, [cols]))
                                return 0

                            lax.fori_loop(0, N_OBS // L, cp_obs, 0)

                            def cp_act(c, _):
                                jj = c * L + lane
                                plsc.store_scatter(
                                    act_stage_t, [jj, b],
                                    plsc.load_gather(drow, [N_OBS + jj]))
                                return 0

                            lax.fori_loop(0, N_ACT // L, cp_act, 0)

                            def cp_nobs(c, _):
                                cols = c * L + lane
                                plsc.store_scatter(
                                    nobs_stage, [b, cols],
                                    plsc.load_gather(
                                        drow, [N_OBS + N_ACT + cols]))
                                return 0

                            lax.fori_loop(0, N_OBS // L, cp_nobs, 0)

                            c0 = N_OBS + N_ACT + N_OBS
                            m0 = lane == 0
                            plsc.store_scatter(
                                rew_o, [b],
                                plsc.load_gather(drow, [jnp.full((L,), c0,
                                                                 jnp.int32)]),
                                mask=m0)
                            plsc.store_scatter(
                                dn_o, [b],
                                plsc.load_gather(drow, [jnp.full((L,), c0 + 1,
                                                                 jnp.int32)]),
                                mask=m0)
                            plsc.store_scatter(
                                tr_o, [b],
                                plsc.load_gather(drow, [jnp.full((L,), c0 + 2,
                                                                 jnp.int32)]),
                                mask=m0)
                        return 0

                    lax.fori_loop(0, L, per_lane, 0)
                return 0

            lax.fori_loop(0, NB, patch_chunk, 0)

            orow = pl.ds(e * BATCH, BATCH)
            pltpu.sync_copy(obs_stage, o_obs.at[orow])
            pltpu.sync_copy(act_stage_t, o_act.at[:, orow])
            pltpu.sync_copy(nobs_stage, o_nobs.at[orow])
            pltpu.sync_copy(rew_o, o_rew.at[orow])
            pltpu.sync_copy(dn_o, o_dn.at[orow])
            pltpu.sync_copy(tr_o, o_tr.at[orow])

    return k


def kernel(observations, actions, rewards, dones, truncations,
           next_observations, obs_data, act_data, next_obs_data, rewards_data,
           dones_data, truncations_data, ptr, sample_idx):
    info = plsc.get_sparse_core_info()
    k = _build_kernel(info.num_cores, info.num_subcores)
    t = jnp.asarray(ptr, jnp.int32) % BUF
    tvec = jnp.full((L,), t, jnp.int32)
    data_comb = jnp.concatenate([
        obs_data, act_data, next_obs_data,
        rewards_data[:, None],
        dones_data[:, None].astype(jnp.float32),
        truncations_data[:, None].astype(jnp.float32),
        jnp.zeros((N_ENV, DROW_PAD - OUT_D), jnp.float32),
    ], axis=1)
    s_obs, s_act_t, s_nobs, s_rew, s_dn, s_tr = k(
        observations.reshape(N_ENV * BUF, N_OBS),
        actions.reshape(N_ENV * BUF // 4, 128),
        next_observations.reshape(N_ENV * BUF, N_OBS),
        rewards.reshape(-1), dones.reshape(-1), truncations.reshape(-1),
        data_comb.reshape(-1), tvec,
        sample_idx.reshape(-1).astype(jnp.int32))
    return jnp.concatenate(
        [s_obs, s_act_t.T, s_nobs, s_rew[:, None], s_dn[:, None],
         s_tr[:, None]],
        axis=-1)
